# baseline (speedup 1.0000x reference)
import functools

import jax
import jax.numpy as jnp
from jax import lax
from jax.experimental import pallas as pl
from jax.experimental.pallas import tpu as pltpu
from jax.experimental.pallas import tpu_sc as plsc

_NC = 2
_NS = 16
_NW = _NC * _NS
_CHUNK = 128


def _make_sc_gather(vocab: int, embed: int, n_idx: int):
  per_w = n_idx // _NW
  n_chunks = per_w // _CHUNK
  mesh = plsc.VectorSubcoreMesh(
      core_axis_name="c", subcore_axis_name="s",
      num_cores=_NC, num_subcores=_NS)

  @functools.partial(
      pl.kernel,
      mesh=mesh,
      compiler_params=pltpu.CompilerParams(use_tc_tiling_on_sc=False),
      out_type=jax.ShapeDtypeStruct((_NW, n_chunks, _CHUNK, embed),
                                    jnp.float32),
      scratch_types=[
          pltpu.VMEM((n_chunks, _CHUNK), jnp.int32),
          pltpu.VMEM((n_chunks, _CHUNK, embed), jnp.float32),
          pltpu.SemaphoreType.DMA,
      ],
  )
  def sc_gather(table_hbm, idx_hbm, out_hbm, idx_v, rows_v, sem):
    wid = lax.axis_index("s") * _NC + lax.axis_index("c")
    pltpu.sync_copy(idx_hbm.at[wid], idx_v)
    copies = [
        pltpu.async_copy(table_hbm.at[idx_v.at[j]], rows_v.at[j], sem)
        for j in range(n_chunks)
    ]
    for c in copies:
      c.wait()
    pltpu.sync_copy(rows_v, out_hbm.at[wid])

  return sc_gather


def _mlp_body(flat_ref, w1_ref, b1_ref, w2_ref, b2_ref, out_ref, h_ref):
  @pl.when(pl.program_id(0) == 0)
  def _():
    pre = jnp.dot(flat_ref[...], w1_ref[...],
                  preferred_element_type=jnp.float32)
    h_ref[...] = jnp.tanh(pre + b1_ref[...])
  out_ref[...] = (
      jnp.dot(h_ref[...], w2_ref[...], preferred_element_type=jnp.float32)
      + b2_ref[...])


def _mlp(flat, W1, b1, W2, b2, v_tile: int, interpret: bool = False):
  batch, feat = flat.shape
  hidden, vocab = W2.shape
  n_tiles = pl.cdiv(vocab, v_tile)
  return pl.pallas_call(
      _mlp_body,
      grid=(n_tiles,),
      in_specs=[
          pl.BlockSpec((batch, feat), lambda i: (0, 0)),
          pl.BlockSpec((feat, hidden), lambda i: (0, 0)),
          pl.BlockSpec((1, hidden), lambda i: (0, 0)),
          pl.BlockSpec((hidden, v_tile), lambda i: (0, i)),
          pl.BlockSpec((1, v_tile), lambda i: (0, i)),
      ],
      out_specs=pl.BlockSpec((batch, v_tile), lambda i: (0, i)),
      out_shape=jax.ShapeDtypeStruct((batch, vocab), jnp.float32),
      scratch_shapes=[pltpu.VMEM((batch, hidden), jnp.float32)],
      interpret=interpret,
  )(flat, W1, b1.reshape(1, hidden), W2, b2.reshape(1, vocab))


def kernel(x, embedding, W1, b1, W2, b2):
  batch, ctx = x.shape
  vocab, embed = embedding.shape
  n_idx = batch * ctx

  per_w = n_idx // _NW
  idx = x.reshape(_NW, per_w // _CHUNK, _CHUNK)
  gathered = _make_sc_gather(vocab, embed, n_idx)(embedding, idx)
  flat = gathered.reshape(batch, ctx * embed)
  return _mlp(flat, W1, b1, W2, b2, v_tile=2048)


# speedup vs baseline: 1.0031x; 1.0031x over previous
import functools

import jax
import jax.numpy as jnp
from jax import lax
from jax.experimental import pallas as pl
from jax.experimental.pallas import tpu as pltpu
from jax.experimental.pallas import tpu_sc as plsc

_NC = 2
_NS = 16
_NW = _NC * _NS
_CHUNK = 128


def _make_sc_gather(vocab: int, embed: int, n_idx: int):
  per_w = n_idx // _NW
  n_chunks = per_w // _CHUNK
  mesh = plsc.VectorSubcoreMesh(
      core_axis_name="c", subcore_axis_name="s",
      num_cores=_NC, num_subcores=_NS)

  @functools.partial(
      pl.kernel,
      mesh=mesh,
      compiler_params=pltpu.CompilerParams(use_tc_tiling_on_sc=False),
      out_type=jax.ShapeDtypeStruct((_NW, n_chunks, _CHUNK, embed),
                                    jnp.float32),
      scratch_types=[
          pltpu.VMEM((n_chunks, _CHUNK), jnp.int32),
          pltpu.VMEM((n_chunks, _CHUNK, embed), jnp.float32),
          pltpu.SemaphoreType.DMA,
      ],
  )
  def sc_gather(table_hbm, idx_hbm, out_hbm, idx_v, rows_v, sem):
    wid = lax.axis_index("s") * _NC + lax.axis_index("c")
    pltpu.sync_copy(idx_hbm.at[wid], idx_v)
    copies = [
        pltpu.async_copy(table_hbm.at[idx_v.at[j]], rows_v.at[j], sem)
        for j in range(n_chunks)
    ]
    for c in copies:
      c.wait()
    pltpu.sync_copy(rows_v, out_hbm.at[wid])

  return sc_gather


def _mlp_body(flat_ref, w1_ref, b1_ref, w2_ref, b2_ref, out_ref, h_ref):
  @pl.when(pl.program_id(0) == 0)
  def _():
    pre = jnp.dot(flat_ref[...], w1_ref[...],
                  preferred_element_type=jnp.float32)
    h_ref[...] = jnp.tanh(pre + b1_ref[...])
  out_ref[...] = (
      jnp.dot(h_ref[...], w2_ref[...], preferred_element_type=jnp.float32)
      + b2_ref[...])


def _mlp(flat, W1, b1, W2, b2, v_tile: int, interpret: bool = False):
  batch, feat = flat.shape
  hidden, vocab = W2.shape
  n_tiles = pl.cdiv(vocab, v_tile)
  return pl.pallas_call(
      _mlp_body,
      grid=(n_tiles,),
      in_specs=[
          pl.BlockSpec((batch, feat), lambda i: (0, 0)),
          pl.BlockSpec((feat, hidden), lambda i: (0, 0)),
          pl.BlockSpec((1, hidden), lambda i: (0, 0)),
          pl.BlockSpec((hidden, v_tile), lambda i: (0, i)),
          pl.BlockSpec((1, v_tile), lambda i: (0, i)),
      ],
      out_specs=pl.BlockSpec((batch, v_tile), lambda i: (0, i)),
      out_shape=jax.ShapeDtypeStruct((batch, vocab), jnp.float32),
      scratch_shapes=[pltpu.VMEM((batch, hidden), jnp.float32)],
      interpret=interpret,
  )(flat, W1, b1.reshape(1, hidden), W2, b2.reshape(1, vocab))


def kernel(x, embedding, W1, b1, W2, b2):
  batch, ctx = x.shape
  vocab, embed = embedding.shape
  n_idx = batch * ctx

  per_w = n_idx // _NW
  idx = x.reshape(_NW, per_w // _CHUNK, _CHUNK)
  gathered = _make_sc_gather(vocab, embed, n_idx)(embedding, idx)
  flat = gathered.reshape(batch, ctx * embed)
  return _mlp(flat, W1, b1, W2, b2, v_tile=4096)
